# baseline (speedup 1.0000x reference)
import functools

import jax
import jax.numpy as jnp
from jax import lax
from jax.experimental import pallas as pl
from jax.experimental.pallas import tpu as pltpu
from jax.experimental.pallas import tpu_sc as plsc

NC = 2
NS = 16
NW = NC * NS
NBUF = 2


@functools.partial(jax.jit, static_argnames=("bsz", "hist"))
def _emb_gather(idx_hmajor, table, bsz, hist):
    d = table.shape[1]
    b_per_w = bsz // NW
    chunk = b_per_w
    lines = chunk * d // 128
    assert hist % NBUF == 0
    mesh = plsc.VectorSubcoreMesh(core_axis_name="c", subcore_axis_name="s")
    rph = (bsz * d) // 128

    @functools.partial(
        pl.kernel,
        mesh=mesh,
        out_type=jax.ShapeDtypeStruct((hist, rph, 128), jnp.float32),
        scratch_types=(
            [pltpu.VMEM((chunk,), jnp.int32) for _ in range(NBUF)]
            + [pltpu.VMEM((chunk, d), jnp.float32) for _ in range(NBUF)]
            + [pltpu.VMEM((lines, 128), jnp.float32) for _ in range(NBUF)]
            + [pltpu.SemaphoreType.DMA for _ in range(2 * NBUF)]
        ),
        compiler_params=pltpu.CompilerParams(use_tc_tiling_on_sc=False),
    )
    def gatherk(idx_hbm, table_hbm, mid_hbm, *scratch):
        idx_v = scratch[:NBUF]
        rows_v = scratch[NBUF:2 * NBUF]
        rpk_v = scratch[2 * NBUF:3 * NBUF]
        gsem = scratch[3 * NBUF:4 * NBUF]
        wsem = scratch[4 * NBUF:]
        wid = lax.axis_index("s") * NC + lax.axis_index("c")
        bbase = wid * b_per_w

        def start_gather(h, b):
            off = pl.multiple_of(h * bsz + bbase, 8)
            pltpu.sync_copy(idx_hbm.at[pl.ds(off, chunk)], idx_v[b])
            pltpu.async_copy(table_hbm.at[idx_v[b]], rows_v[b], gsem[b])

        for b in range(NBUF):
            start_gather(b, b)

        rpw = 128 // d

        def group(j, _):
            for b in range(NBUF):
                h = j * NBUF + b
                pltpu.make_async_copy(
                    table_hbm.at[idx_v[b]], rows_v[b], gsem[b]
                ).wait()

                def repack(i0, _):
                    for k in range(8):
                        rr = (8 // rpw) * i0 + (k // rpw)
                        for c0 in range(0, d, 16):
                            cc = (k % rpw) * d + c0
                            rpk_v[b].at[rr][pl.ds(cc, 16)] = (
                                rows_v[b].at[8 * i0 + k][pl.ds(c0, 16)]
                            )
                    return 0

                lax.fori_loop(0, chunk // 8, repack, 0)
                pltpu.async_copy(
                    rpk_v[b],
                    mid_hbm.at[h, pl.ds(wid * lines, lines)],
                    wsem[b],
                ).wait()

                @pl.when(h + NBUF < hist)
                def _():
                    start_gather(h + NBUF, b)
            return 0

        lax.fori_loop(0, hist // NBUF, group, 0)

    return gatherk(idx_hmajor, table)


@functools.partial(jax.jit, static_argnames=("bsz", "hist", "d"))
def _transpose_out(mid, bsz, hist, d):
    cblk = 128
    nblk = bsz // cblk
    groups = hist * nblk
    gpw = groups // NW
    assert gpw % NBUF == 0
    rpb = cblk * d // 128
    mesh = plsc.VectorSubcoreMesh(core_axis_name="c", subcore_axis_name="s")

    @functools.partial(
        pl.kernel,
        mesh=mesh,
        out_type=jax.ShapeDtypeStruct((hist, d, bsz), jnp.float32),
        scratch_types=(
            [pltpu.VMEM((rpb, 128), jnp.float32) for _ in range(NBUF)]
            + [pltpu.VMEM((d, cblk), jnp.float32) for _ in range(NBUF)]
            + [pltpu.SemaphoreType.DMA for _ in range(NBUF)]
            + [pltpu.SemaphoreType.DMA for _ in range(NBUF)]
        ),
        compiler_params=pltpu.CompilerParams(use_tc_tiling_on_sc=True, needs_layout_passes=False),
    )
    def transk(mid_hbm, out_hbm, *scratch):
        in_v = scratch[:NBUF]
        tr_v = scratch[NBUF:2 * NBUF]
        rsem = scratch[2 * NBUF:3 * NBUF]
        wsem = scratch[3 * NBUF:]
        wid = lax.axis_index("s") * NC + lax.axis_index("c")
        lane = lax.iota(jnp.int32, 16)
        rpl = 128 // d
        rowv = [(c0 + lane) // rpl for c0 in range(0, cblk, 16)]
        colb = [((c0 + lane) % rpl) * d for c0 in range(0, cblk, 16)]

        def start_read(g, b):
            grp = wid * gpw + g
            h = grp // nblk
            blk = grp % nblk
            pltpu.async_copy(
                mid_hbm.at[h, pl.ds(blk * rpb, rpb)], in_v[b], rsem[b]
            )

        for b in range(NBUF):
            start_read(b, b)

        def group(j, _):
            for b in range(NBUF):
                g = j * NBUF + b
                grp = wid * gpw + g
                h = grp // nblk
                blk = grp % nblk
                pltpu.make_async_copy(
                    mid_hbm.at[h, pl.ds(blk * rpb, rpb)], in_v[b], rsem[b]
                ).wait()
                for ci, c0 in enumerate(range(0, cblk, 16)):
                    for dd in range(d):
                        v = plsc.load_gather(
                            in_v[b], [rowv[ci], colb[ci] + dd]
                        )
                        tr_v[b].at[dd][pl.ds(c0, 16)] = v
                for dt in range(d // 8):
                    pltpu.async_copy(
                        tr_v[b].at[pl.ds(8 * dt, 8)],
                        out_hbm.at[h, pl.ds(8 * dt, 8), pl.ds(blk * cblk, cblk)],
                        wsem[b],
                    )
                for dt in range(d // 8):
                    pltpu.make_async_copy(
                        tr_v[b].at[pl.ds(8 * dt, 8)],
                        out_hbm.at[h, pl.ds(8 * dt, 8), pl.ds(blk * cblk, cblk)],
                        wsem[b],
                    ).wait()

                @pl.when(g + NBUF < gpw)
                def _():
                    start_read(g + NBUF, b)
            return 0

        lax.fori_loop(0, gpw // NBUF, group, 0)

    return transk(mid)


def kernel(input_ids, table):
    b, h = input_ids.shape
    d = table.shape[1]
    idx_hmajor = jnp.transpose(input_ids, (1, 0)).reshape(h * b)
    mid = _emb_gather(idx_hmajor, table, bsz=b, hist=h)
    out = _transpose_out(mid, bsz=b, hist=h, d=d)
    return jnp.transpose(out, (2, 0, 1))


# speedup vs baseline: 1.0312x; 1.0312x over previous
import functools

import jax
import jax.numpy as jnp
from jax import lax
from jax.experimental import pallas as pl
from jax.experimental.pallas import tpu as pltpu
from jax.experimental.pallas import tpu_sc as plsc

NC = 2
NS = 16
NW = NC * NS
NBUF = 2


@functools.partial(jax.jit, static_argnames=("bsz", "hist"))
def _emb_gather(idx_hmajor, table, bsz, hist):
    d = table.shape[1]
    b_per_w = bsz // NW
    chunk = b_per_w
    lines = chunk * d // 128
    assert hist % NBUF == 0
    mesh = plsc.VectorSubcoreMesh(core_axis_name="c", subcore_axis_name="s")
    rph = (bsz * d) // 128

    @functools.partial(
        pl.kernel,
        mesh=mesh,
        out_type=jax.ShapeDtypeStruct((hist, rph, 128), jnp.float32),
        scratch_types=(
            [pltpu.VMEM((chunk,), jnp.int32) for _ in range(NBUF)]
            + [pltpu.VMEM((chunk, d), jnp.float32) for _ in range(NBUF)]
            + [pltpu.VMEM((lines, 128), jnp.float32) for _ in range(NBUF)]
            + [pltpu.SemaphoreType.DMA for _ in range(2 * NBUF)]
        ),
        compiler_params=pltpu.CompilerParams(use_tc_tiling_on_sc=False),
    )
    def gatherk(idx_hbm, table_hbm, mid_hbm, *scratch):
        idx_v = scratch[:NBUF]
        rows_v = scratch[NBUF:2 * NBUF]
        rpk_v = scratch[2 * NBUF:3 * NBUF]
        gsem = scratch[3 * NBUF:4 * NBUF]
        wsem = scratch[4 * NBUF:]
        wid = lax.axis_index("s") * NC + lax.axis_index("c")
        bbase = wid * b_per_w

        def start_gather(h, b):
            off = pl.multiple_of(h * bsz + bbase, 8)
            pltpu.sync_copy(idx_hbm.at[pl.ds(off, chunk)], idx_v[b])
            pltpu.async_copy(table_hbm.at[idx_v[b]], rows_v[b], gsem[b])

        for b in range(NBUF):
            start_gather(b, b)

        rpw = 128 // d

        def group(j, _):
            for b in range(NBUF):
                h = j * NBUF + b
                pltpu.make_async_copy(
                    table_hbm.at[idx_v[b]], rows_v[b], gsem[b]
                ).wait()

                def repack(i0, _):
                    for k in range(8):
                        rr = (8 // rpw) * i0 + (k // rpw)
                        for c0 in range(0, d, 16):
                            cc = (k % rpw) * d + c0
                            rpk_v[b].at[rr][pl.ds(cc, 16)] = (
                                rows_v[b].at[8 * i0 + k][pl.ds(c0, 16)]
                            )
                    return 0

                lax.fori_loop(0, chunk // 8, repack, 0)
                pltpu.async_copy(
                    rpk_v[b],
                    mid_hbm.at[h, pl.ds(wid * lines, lines)],
                    wsem[b],
                ).wait()

                @pl.when(h + NBUF < hist)
                def _():
                    start_gather(h + NBUF, b)
            return 0

        lax.fori_loop(0, hist // NBUF, group, 0)

    return gatherk(idx_hmajor, table)


@functools.partial(jax.jit, static_argnames=("bsz", "hist", "d"))
def _transpose_out(mid, bsz, hist, d):
    cblk = 128
    nblk = bsz // cblk
    groups = hist * nblk
    gpw = groups // NW
    assert gpw % NBUF == 0
    rpb = cblk * d // 128
    mesh = plsc.VectorSubcoreMesh(core_axis_name="c", subcore_axis_name="s")

    @functools.partial(
        pl.kernel,
        mesh=mesh,
        out_type=jax.ShapeDtypeStruct((hist, d, bsz), jnp.float32),
        scratch_types=(
            [pltpu.VMEM((rpb, 128), jnp.float32) for _ in range(NBUF)]
            + [pltpu.VMEM((d, cblk), jnp.float32) for _ in range(NBUF)]
            + [pltpu.SemaphoreType.DMA for _ in range(NBUF)]
            + [pltpu.SemaphoreType.DMA for _ in range(NBUF)]
        ),
        compiler_params=pltpu.CompilerParams(use_tc_tiling_on_sc=True, needs_layout_passes=False),
    )
    def transk(mid_hbm, out_hbm, *scratch):
        in_v = scratch[:NBUF]
        tr_v = scratch[NBUF:2 * NBUF]
        rsem = scratch[2 * NBUF:3 * NBUF]
        wsem = scratch[3 * NBUF:]
        wid = lax.axis_index("s") * NC + lax.axis_index("c")
        lane = lax.iota(jnp.int32, 16)
        rpl = 128 // d
        rowv = [(c0 + lane) // rpl for c0 in range(0, cblk, 16)]
        colb = [((c0 + lane) % rpl) * d for c0 in range(0, cblk, 16)]

        def start_read(g, b):
            grp = wid * gpw + g
            h = grp // nblk
            blk = grp % nblk
            pltpu.async_copy(
                mid_hbm.at[h, pl.ds(blk * rpb, rpb)], in_v[b], rsem[b]
            )

        for b in range(NBUF):
            start_read(b, b)

        def group(j, _):
            for b in range(NBUF):
                g = j * NBUF + b
                grp = wid * gpw + g
                h = grp // nblk
                blk = grp % nblk
                pltpu.make_async_copy(
                    mid_hbm.at[h, pl.ds(blk * rpb, rpb)], in_v[b], rsem[b]
                ).wait()
                @pl.when(g >= NBUF)
                def _():
                    pltpu.make_async_copy(
                        tr_v[b],
                        out_hbm.at[h, :, pl.ds(blk * cblk, cblk)],
                        wsem[b],
                    ).wait()

                for ci, c0 in enumerate(range(0, cblk, 16)):
                    for dd in range(d):
                        v = plsc.load_gather(
                            in_v[b], [rowv[ci], colb[ci] + dd]
                        )
                        tr_v[b].at[dd][pl.ds(c0, 16)] = v
                pltpu.async_copy(
                    tr_v[b],
                    out_hbm.at[h, :, pl.ds(blk * cblk, cblk)],
                    wsem[b],
                )

                @pl.when(g + NBUF < gpw)
                def _():
                    start_read(g + NBUF, b)
            return 0

        lax.fori_loop(0, gpw // NBUF, group, 0)
        for b in range(NBUF):
            pltpu.make_async_copy(
                tr_v[b], out_hbm.at[0, :, pl.ds(0, cblk)], wsem[b]
            ).wait()

    return transk(mid)


def kernel(input_ids, table):
    b, h = input_ids.shape
    d = table.shape[1]
    idx_hmajor = jnp.transpose(input_ids, (1, 0)).reshape(h * b)
    mid = _emb_gather(idx_hmajor, table, bsz=b, hist=h)
    out = _transpose_out(mid, bsz=b, hist=h, d=d)
    return jnp.transpose(out, (2, 0, 1))


# speedup vs baseline: 1.3332x; 1.2928x over previous
import functools

import jax
import jax.numpy as jnp
from jax import lax
from jax.experimental import pallas as pl
from jax.experimental.pallas import tpu as pltpu
from jax.experimental.pallas import tpu_sc as plsc

NC = 2
NS = 16
NW = NC * NS
NBUF = 2


@functools.partial(jax.jit, static_argnames=("bblk", "bsz", "hist"))
def _emb_lookup(idx_flat, table, bblk, bsz, hist):
    d = table.shape[1]
    chunk = bblk * hist
    b_per_w = bsz // NW
    nchunk = b_per_w // bblk
    assert nchunk % NBUF == 0
    mesh = plsc.VectorSubcoreMesh(core_axis_name="c", subcore_axis_name="s")

    @functools.partial(
        pl.kernel,
        mesh=mesh,
        out_type=jax.ShapeDtypeStruct((bsz, hist, d), jnp.float32),
        scratch_types=(
            [pltpu.VMEM((chunk,), jnp.int32) for _ in range(NBUF)]
            + [pltpu.VMEM((chunk, d), jnp.float32) for _ in range(NBUF)]
            + [pltpu.SemaphoreType.DMA for _ in range(2 * NBUF)]
        ),
        compiler_params=pltpu.CompilerParams(use_tc_tiling_on_sc=False),
    )
    def emb(idx_hbm, table_hbm, out_hbm, *scratch):
        idx_v = scratch[:NBUF]
        rows_v = scratch[NBUF:2 * NBUF]
        gsem = scratch[2 * NBUF:3 * NBUF]
        wsem = scratch[3 * NBUF:]
        wid = lax.axis_index("s") * NC + lax.axis_index("c")
        bbase = wid * b_per_w
        fbase = bbase * hist

        def start_gather(g, b):
            off = pl.multiple_of(fbase + g * chunk, 8)
            pltpu.sync_copy(idx_hbm.at[pl.ds(off, chunk)], idx_v[b])
            pltpu.async_copy(table_hbm.at[idx_v[b]], rows_v[b], gsem[b])

        for b in range(NBUF):
            start_gather(b, b)

        def group(j, _):
            for b in range(NBUF):
                g = j * NBUF + b
                pltpu.make_async_copy(
                    table_hbm.at[idx_v[b]], rows_v[b], gsem[b]
                ).wait()
                b0 = bbase + g * bblk
                for j2 in range(bblk):
                    pltpu.async_copy(
                        rows_v[b].at[pl.ds(j2 * hist, hist)],
                        out_hbm.at[b0 + j2],
                        wsem[b],
                    )
                for j2 in range(bblk):
                    pltpu.make_async_copy(
                        rows_v[b].at[pl.ds(j2 * hist, hist)],
                        out_hbm.at[b0 + j2],
                        wsem[b],
                    ).wait()

                @pl.when(g + NBUF < nchunk)
                def _():
                    start_gather(g + NBUF, b)
            return 0

        lax.fori_loop(0, nchunk // NBUF, group, 0)

    return emb(idx_flat, table)


def kernel(input_ids, table):
    b, h = input_ids.shape
    return _emb_lookup(input_ids.reshape(b * h), table, bblk=32, bsz=b, hist=h)
